# baseline (speedup 1.0000x reference)
import jax
import jax.numpy as jnp
from jax import lax
from jax.experimental import pallas as pl
from jax.experimental.pallas import tpu as pltpu

_B, _N, _H = 8, 128, 768
_NEG_INF = float("-inf")
_BIG = 1 << 30


def _merge_body(set1_ref, set2_ref, out_ref, sim_ref):
    lane_iota = lax.broadcasted_iota(jnp.int32, (_N, _N), 1)
    sub_iota = lax.broadcasted_iota(jnp.int32, (_N, _N), 0)
    ij_iota = sub_iota * _N + lane_iota
    b_iota = lax.broadcasted_iota(jnp.int32, (_B, _N), 0)
    r_iota = lax.broadcasted_iota(jnp.int32, (_B, _N), 1)

    def _rownorm(x):
        xx = x * x
        a = xx[:, 0:128]
        for c in range(1, 6):
            a = a + xx[:, 128 * c:128 * (c + 1)]
        v = a[:, 0:8]
        for g in range(1, 16):
            v = v + a[:, 8 * g:8 * (g + 1)]
        w = v[:, 0:4] + v[:, 4:8]
        t = w[:, 0:2] + w[:, 2:4]
        return jnp.sqrt(t[:, 0:1] + t[:, 1:2])

    for b in range(_B):
        x1 = set1_ref[b]
        x2 = set2_ref[b]
        s1 = x1 / jnp.maximum(_rownorm(x1), 1e-12)
        s2 = x2 / jnp.maximum(_rownorm(x2), 1e-12)
        sim_ref[b] = lax.dot_general(s1, s2, (((1,), (1,)), ((), ())),
                                     preferred_element_type=jnp.float32)

    def step(r, ch):
        for b in range(_B):
            s = sim_ref[b]
            m = jnp.max(jnp.max(s, axis=1, keepdims=True), axis=0,
                        keepdims=True)
            cand = jnp.where(s == m, ij_iota, _BIG)
            chosen = jnp.min(jnp.min(cand, axis=1, keepdims=True), axis=0,
                             keepdims=True)
            sim_ref[b] = jnp.where(ij_iota == chosen, _NEG_INF, s)
            upd = (b_iota == b) & (r_iota == r)
            ch = jnp.where(upd, jnp.broadcast_to(chosen, (_B, _N)), ch)
        return ch

    ch = lax.fori_loop(0, _N, step, jnp.zeros((_B, _N), jnp.int32))
    i_idx = ch // _N
    j_idx = ch % _N

    for b in range(_B):
        oht1 = jnp.where(sub_iota == i_idx[b:b + 1, :], jnp.float32(0.5),
                         jnp.float32(0.0))
        oht2 = jnp.where(sub_iota == j_idx[b:b + 1, :], jnp.float32(0.5),
                         jnp.float32(0.0))
        g1 = lax.dot_general(oht1, set1_ref[b], (((0,), (0,)), ((), ())),
                             precision=lax.Precision.HIGHEST,
                             preferred_element_type=jnp.float32)
        g2 = lax.dot_general(oht2, set2_ref[b], (((0,), (0,)), ((), ())),
                             precision=lax.Precision.HIGHEST,
                             preferred_element_type=jnp.float32)
        out_ref[b] = g1 + g2


def _merged_tokens(set1, set2):
    return pl.pallas_call(
        _merge_body,
        out_shape=jax.ShapeDtypeStruct((_B, _N, _H), jnp.float32),
        scratch_shapes=[
            pltpu.VMEM((_B, _N, _N), jnp.float32),
        ],
    )(set1, set2)


def kernel(K):
    batch, num_tokens, hidden = K.shape
    Kr = K.reshape(batch, num_tokens // 2, 2, hidden)
    set1 = Kr[:, :, 0, :]
    set2 = Kr[:, :, 1, :]
    merged = _merged_tokens(set1, set2)
    return (merged, num_tokens // 2)
